# Initial kernel scaffold; baseline (speedup 1.0000x reference)
import jax
import jax.numpy as jnp
from jax.experimental import pallas as pl


def kernel(x, adj_dist, adj_direct, wind_mean, wind_std, W1, b1, W2, b2, W3, b3):
    raise NotImplementedError("write your pallas kernel here")


# speedup vs baseline: 3.9925x; 3.9925x over previous
import jax
import jax.numpy as jnp
from jax.experimental import pallas as pl

_B, _NC, _IN = 32, 128, 12
_EH, _EO, _OUT = 32, 30, 12
_GC = 8
_NCH = _NC // _GC


def _stats_kernel(d_ref, g_ref, out_ref):
    n = _B * _NC
    d = d_ref[...].reshape(n, _NC)
    g = g_ref[...].reshape(n, _NC)
    md = jnp.mean(d, axis=0)
    vd = jnp.sum((d - md[None, :]) ** 2, axis=0) / (n - 1)
    rd = jax.lax.rsqrt(vd)
    mg = jnp.mean(g, axis=0)
    vg = jnp.sum((g - mg[None, :]) ** 2, axis=0) / (n - 1)
    rg = jax.lax.rsqrt(vg)
    out_ref[...] = jnp.concatenate(
        [rd[None, :], (-md * rd)[None, :], rg[None, :], (-mg * rg)[None, :]],
        axis=0,
    )


def _main_kernel(stats_ref, at_ref, bt_ref, ub_ref, vb_ref, wb_ref, b1b_ref,
                 bd_ref, b2b_ref, w3_ref, b3b_ref, x_ref, d_ref, g_ref,
                 out_ref):
    xb = x_ref[0]
    d = d_ref[0]
    g = g_ref[0]
    st = stats_ref[...]

    adjf = (d != 0.0).astype(jnp.float32)
    dn = d * st[0:1, :] + st[1:2, :]
    gn = g * st[2:3, :] + st[3:4, :]

    s = jnp.dot(xb, at_ref[...])
    t = jnp.dot(xb, bt_ref[...])
    tw = t.T + wb_ref[...]

    pre = (adjf[:, None, :] * (s[:, :, None] + tw[None, :, :])
           + dn[:, None, :] * ub_ref[...][None, :, :]
           + gn[:, None, :] * vb_ref[...][None, :, :]
           + b1b_ref[...][None, :, :])
    h = jax.nn.sigmoid(pre)

    bd = bd_ref[...]
    b2b = b2b_ref[...]
    add_acc = jnp.zeros((_GC * _EO, _NC), jnp.float32)
    subs = []
    for c in range(_NCH):
        hc = h[c * _GC:(c + 1) * _GC].reshape(_GC * _EH, _NC)
        ec = jax.nn.sigmoid(jnp.dot(bd, hc) + b2b)
        add_acc = add_acc + ec
        subs.append(jnp.sum(ec.reshape(_GC, _EO, _NC), axis=2))
    add = jnp.sum(add_acc.reshape(_GC, _EO, _NC), axis=0)
    sub = jnp.concatenate(subs, axis=0)
    cmat = add - sub.T

    o = jax.nn.sigmoid(jnp.dot(w3_ref[...], cmat) + b3b_ref[...])
    out_ref[0] = o.T


def kernel(x, adj_dist, adj_direct, wind_mean, wind_std, W1, b1, W2, b2, W3,
           b3):
    del wind_mean, wind_std
    f32 = jnp.float32

    at = W1[:, 0:12].T
    bt = W1[:, 13:25].T
    u = W1[:, 26]
    v = W1[:, 28]
    w = (W1[:, 12] + W1[:, 25] + W1[:, 27] + W1[:, 29] + W1[:, 30]
         + W1[:, 31])
    ub = jnp.broadcast_to(u[:, None], (_EH, _NC))
    vb = jnp.broadcast_to(v[:, None], (_EH, _NC))
    wb = jnp.broadcast_to(w[:, None], (_EH, _NC))
    b1b = jnp.broadcast_to(b1[:, None], (_EH, _NC))
    eye = jnp.eye(_GC, dtype=f32)
    bd = jnp.einsum("gh,ok->gohk", eye, W2).reshape(_GC * _EO, _GC * _EH)
    b2b = jnp.broadcast_to(jnp.tile(b2, _GC)[:, None], (_GC * _EO, _NC))
    b3b = jnp.broadcast_to(b3[:, None], (_OUT, _NC))

    stats = pl.pallas_call(
        _stats_kernel,
        out_shape=jax.ShapeDtypeStruct((4, _NC), f32),
    )(adj_dist, adj_direct)

    const2 = lambda shape: pl.BlockSpec(shape, lambda b: (0, 0))
    out = pl.pallas_call(
        _main_kernel,
        grid=(_B,),
        in_specs=[
            const2((4, _NC)),
            const2((_IN, _EH)),
            const2((_IN, _EH)),
            const2((_EH, _NC)),
            const2((_EH, _NC)),
            const2((_EH, _NC)),
            const2((_EH, _NC)),
            const2((_GC * _EO, _GC * _EH)),
            const2((_GC * _EO, _NC)),
            const2((_OUT, _EO)),
            const2((_OUT, _NC)),
            pl.BlockSpec((1, _NC, _IN), lambda b: (b, 0, 0)),
            pl.BlockSpec((1, _NC, _NC), lambda b: (b, 0, 0)),
            pl.BlockSpec((1, _NC, _NC), lambda b: (b, 0, 0)),
        ],
        out_specs=pl.BlockSpec((1, _NC, _OUT), lambda b: (b, 0, 0)),
        out_shape=jax.ShapeDtypeStruct((_B, _NC, _OUT), f32),
    )(stats, at, bt, ub, vb, wb, b1b, bd, b2b, W3, b3b, x, adj_dist,
      adj_direct)
    return out


# speedup vs baseline: 5.4398x; 1.3625x over previous
import jax
import jax.numpy as jnp
from jax.experimental import pallas as pl

_B, _NC, _IN = 32, 128, 12
_EH, _EO, _OUT = 32, 30, 12
_GC = 8
_NCH = _NC // _GC


def _prep_kernel(d_ref, g_ref, x_ref, w1_ref, b1_ref, b2_ref, b3_ref,
                 adjf_ref, dn_ref, gn_ref, s_ref, tw_ref, kb_ref, b2b_ref,
                 b3b_ref):
    n = _B * _NC
    bf = jnp.bfloat16
    d = d_ref[...].reshape(n, _NC)
    g = g_ref[...].reshape(n, _NC)
    md = jnp.mean(d, axis=0)
    vd = jnp.sum((d - md[None, :]) ** 2, axis=0) / (n - 1)
    rd = jax.lax.rsqrt(vd)
    mg = jnp.mean(g, axis=0)
    vg = jnp.sum((g - mg[None, :]) ** 2, axis=0) / (n - 1)
    rg = jax.lax.rsqrt(vg)

    adjf_ref[...] = (d != 0.0).astype(bf).reshape(_B, _NC, _NC)
    dn_ref[...] = ((d * rd[None, :] - (md * rd)[None, :])
                   .astype(bf).reshape(_B, _NC, _NC))
    gn_ref[...] = ((g * rg[None, :] - (mg * rg)[None, :])
                   .astype(bf).reshape(_B, _NC, _NC))

    w1 = w1_ref[...]
    at = w1[:, 0:12].T
    bt = w1[:, 13:25].T
    u = w1[:, 26:27]
    v = w1[:, 28:29]
    w = (w1[:, 12:13] + w1[:, 25:26] + w1[:, 27:28] + w1[:, 29:30]
         + w1[:, 30:31] + w1[:, 31:32])
    b1c = b1_ref[...].T
    kb_ref[...] = jnp.concatenate(
        [jnp.broadcast_to(u, (_EH, _NC)),
         jnp.broadcast_to(v, (_EH, _NC)),
         jnp.broadcast_to(b1c, (_EH, _NC))], axis=0).astype(bf)

    x2 = x_ref[...].reshape(n, _IN)
    s_ref[...] = jnp.dot(x2, at).astype(bf).reshape(_B, _NC, _EH)
    t3 = jnp.dot(x2, bt).reshape(_B, _NC, _EH)
    wb = jnp.broadcast_to(w, (_EH, _NC))
    tw_ref[...] = (jnp.transpose(t3, (0, 2, 1)) + wb[None, :, :]).astype(bf)

    b2m = jnp.broadcast_to(b2_ref[...].T, (_EO, _NC))
    b2b_ref[...] = jnp.broadcast_to(
        b2m[:, None, :], (_EO, _GC, _NC)).reshape(_EO * _GC, _NC)
    b3b_ref[...] = jnp.broadcast_to(b3_ref[...].T, (_OUT, _NC))


def _main_kernel(kb_ref, bd_ref, b2b_ref, w3_ref, b3b_ref, adjf_ref, dn_ref,
                 gn_ref, s_ref, tw_ref, out_ref):
    adjf = adjf_ref[0]
    dn = dn_ref[0]
    gn = gn_ref[0]
    sb = s_ref[0]
    tw = tw_ref[0]
    kb = kb_ref[...]
    ub = kb[0:_EH]
    vb = kb[_EH:2 * _EH]
    b1b = kb[2 * _EH:3 * _EH]

    pre = (adjf[:, None, :] * (sb[:, :, None] + tw[None, :, :])
           + dn[:, None, :] * ub[None, :, :]
           + gn[:, None, :] * vb[None, :, :]
           + b1b[None, :, :])
    h = jax.nn.sigmoid(pre)

    bd = bd_ref[...]
    b2b = b2b_ref[...]
    add_acc = jnp.zeros((_EO * _GC, _NC), jnp.float32)
    subs = []
    for c in range(_NCH):
        hc = h[c * _GC:(c + 1) * _GC].reshape(_GC * _EH, _NC)
        ec = jax.nn.sigmoid(
            jnp.dot(bd, hc, preferred_element_type=jnp.float32) + b2b)
        add_acc = add_acc + ec
        subs.append(jnp.sum(ec.reshape(_EO, _GC, _NC), axis=2))
    add = jnp.sum(add_acc.reshape(_EO, _GC, _NC), axis=1)
    sub = jnp.concatenate(subs, axis=1)
    cmat = add - sub

    o = jax.nn.sigmoid(jnp.dot(w3_ref[...], cmat) + b3b_ref[...])
    out_ref[0] = o.T


def kernel(x, adj_dist, adj_direct, wind_mean, wind_std, W1, b1, W2, b2, W3,
           b3):
    del wind_mean, wind_std
    f32 = jnp.float32
    bf = jnp.bfloat16

    eye = jnp.eye(_GC, dtype=f32)
    bd = jnp.einsum("gh,ok->oghk", eye, W2).reshape(
        _EO * _GC, _GC * _EH).astype(bf)

    prep = pl.pallas_call(
        _prep_kernel,
        out_shape=(
            jax.ShapeDtypeStruct((_B, _NC, _NC), bf),
            jax.ShapeDtypeStruct((_B, _NC, _NC), bf),
            jax.ShapeDtypeStruct((_B, _NC, _NC), bf),
            jax.ShapeDtypeStruct((_B, _NC, _EH), bf),
            jax.ShapeDtypeStruct((_B, _EH, _NC), bf),
            jax.ShapeDtypeStruct((3 * _EH, _NC), bf),
            jax.ShapeDtypeStruct((_EO * _GC, _NC), f32),
            jax.ShapeDtypeStruct((_OUT, _NC), f32),
        ),
    )(adj_dist, adj_direct, x, W1, b1.reshape(1, _EH), b2.reshape(1, _EO),
      b3.reshape(1, _OUT))
    adjf, dn, gn, s_all, tw_all, kb, b2b, b3b = prep

    const2 = lambda shape: pl.BlockSpec(shape, lambda b: (0, 0))
    out = pl.pallas_call(
        _main_kernel,
        grid=(_B,),
        in_specs=[
            const2((3 * _EH, _NC)),
            const2((_EO * _GC, _GC * _EH)),
            const2((_EO * _GC, _NC)),
            const2((_OUT, _EO)),
            const2((_OUT, _NC)),
            pl.BlockSpec((1, _NC, _NC), lambda b: (b, 0, 0)),
            pl.BlockSpec((1, _NC, _NC), lambda b: (b, 0, 0)),
            pl.BlockSpec((1, _NC, _NC), lambda b: (b, 0, 0)),
            pl.BlockSpec((1, _NC, _EH), lambda b: (b, 0, 0)),
            pl.BlockSpec((1, _EH, _NC), lambda b: (b, 0, 0)),
        ],
        out_specs=pl.BlockSpec((1, _NC, _OUT), lambda b: (b, 0, 0)),
        out_shape=jax.ShapeDtypeStruct((_B, _NC, _OUT), f32),
    )(kb, bd, b2b, W3, b3b, adjf, dn, gn, s_all, tw_all)
    return out
